# baseline (speedup 1.0000x reference)
import jax
import jax.numpy as jnp
from jax import lax
from jax.experimental import pallas as pl
from jax.experimental.pallas import tpu as pltpu
from jax.experimental.pallas import tpu_sc as plsc

_BATCH = 4096
_E = 200
_D = 64
_NC = 2
_NS = 16
_NW = _NC * _NS
_B_PER_W = _BATCH // _NW
_E_PAD = 208


def _body(x_hbm, tab_hbm, const_hbm, out_hbm, idx_v, safe_v, midx_v, buf_v,
          sem0, sem1):
    wid = lax.axis_index("s") * _NC + lax.axis_index("c")
    b0 = wid * _B_PER_W

    def batch_step(i, carry):
        b = b0 + i
        pltpu.sync_copy(x_hbm.at[pl.ds(b * _E, _E)], idx_v.at[pl.ds(0, _E)])

        def cvec(j, c):
            o = pl.multiple_of(j * 16, 16)
            v = idx_v[pl.ds(o, 16)]
            m = v != -1
            safe_v[pl.ds(o, 16)] = jnp.where(m, v, 0)
            midx_v[pl.ds(o, 16)] = jnp.where(m, 1, 0)
            return c

        lax.fori_loop(0, _E // 16, cvec, 0)
        lane = lax.iota(jnp.int32, 16)
        v = idx_v[pl.ds(192, 16)]
        m = (lane < (_E - 192)) & (v != -1)
        safe_v[pl.ds(192, 16)] = jnp.where(m, v, 0)
        midx_v[pl.ds(192, 16)] = jnp.where(m, 1, 0)

        cps = []
        for (lo, ln) in ((0, 128), (128, 80)):
            cps.append(pltpu.make_async_copy(
                tab_hbm.at[safe_v.at[pl.ds(lo, ln)]],
                buf_v.at[0, pl.ds(lo, ln)], sem0))
            cps.append(pltpu.make_async_copy(
                const_hbm.at[midx_v.at[pl.ds(lo, ln)]],
                buf_v.at[1, pl.ds(lo, ln)], sem1))
        for cp in cps:
            cp.start()
        for cp in cps:
            cp.wait()

        pltpu.sync_copy(buf_v.at[0, pl.ds(0, _E)],
                        out_hbm.at[pl.ds(b * 2 * _E, _E)])
        pltpu.sync_copy(buf_v.at[1, pl.ds(0, _E)],
                        out_hbm.at[pl.ds(b * 2 * _E + _E, _E)])
        return carry

    lax.fori_loop(0, _B_PER_W, batch_step, 0)


def kernel(x, entity_vectors):
    const_tab = jnp.concatenate(
        [jnp.zeros((1, _D), jnp.float32), jnp.ones((1, _D), jnp.float32)])
    mesh = plsc.VectorSubcoreMesh(core_axis_name="c", subcore_axis_name="s")
    run = pl.kernel(
        _body,
        out_type=jax.ShapeDtypeStruct((_BATCH * 2 * _E, _D), jnp.float32),
        mesh=mesh,
        compiler_params=pltpu.CompilerParams(use_tc_tiling_on_sc=False),
        scratch_types=[
            pltpu.VMEM((_E_PAD,), jnp.int32),
            pltpu.VMEM((_E_PAD,), jnp.int32),
            pltpu.VMEM((_E_PAD,), jnp.int32),
            pltpu.VMEM((2, _E_PAD, _D), jnp.float32),
            pltpu.SemaphoreType.DMA,
            pltpu.SemaphoreType.DMA,
        ],
    )
    out = run(x.reshape(-1), entity_vectors, const_tab)
    return out.reshape(_BATCH, 2, _E, _D)


# speedup vs baseline: 9.4388x; 9.4388x over previous
import jax
import jax.numpy as jnp
from jax import lax
from jax.experimental import pallas as pl
from jax.experimental.pallas import tpu as pltpu
from jax.experimental.pallas import tpu_sc as plsc

_BATCH = 4096
_E = 200
_D = 64
_NC = 2
_NS = 16
_NW = _NC * _NS
_BPW = _BATCH // _NW
_G = 2
_GL = _G * _E
_NGRP = _BPW // _G
_CH = 80
_NCH = _GL // _CH


def _body(x_hbm, tab_hbm, const_hbm, out_hbm,
          idxraw, safe, midx, buf, mrow, ones_v, aflag, sem0, sem1, semm):
    wid = lax.axis_index("s") * _NC + lax.axis_index("c")
    b0 = wid * _BPW
    sems = (sem0, sem1)

    ones16 = jnp.ones((16,), jnp.float32)

    def fill(e, c):
        for k in range(_D // 16):
            ones_v[e, pl.ds(k * 16, 16)] = ones16
        return c

    lax.fori_loop(0, _E, fill, 0)

    def prep(g, p):
        bb = b0 + g * _G
        pltpu.sync_copy(x_hbm.at[pl.ds(bb * _E, _GL)], idxraw.at[p])

        def cvec(j, acc):
            o = pl.multiple_of(j * 16, 16)
            v = idxraw[p, pl.ds(o, 16)]
            valid = v != -1
            mi = jnp.where(valid, 1, 0)
            safe[p, pl.ds(o, 16)] = jnp.where(valid, v, 0)
            midx[p, pl.ds(o, 16)] = mi
            return acc + mi

        acc = lax.fori_loop(0, _GL // 16, cvec, jnp.zeros((16,), jnp.int32))
        total = acc[0]
        for l in range(1, 16):
            total = total + acc[l]
        aflag[p] = total - _GL
        for c in range(_NCH):
            pltpu.make_async_copy(
                tab_hbm.at[safe.at[p, pl.ds(c * _CH, _CH)]],
                buf.at[p, pl.ds(c * _CH, _CH)], sems[p]).start()

    def outcopy(g, p):
        bb = b0 + g * _G
        pltpu.make_async_copy(tab_hbm.at[pl.ds(0, _GL)], buf.at[p],
                              sems[p]).wait()
        a = aflag[p]

        @pl.when(a == 0)
        def _():
            for k in range(_G):
                pltpu.sync_copy(ones_v,
                                out_hbm.at[pl.ds((bb + k) * 2 * _E + _E, _E)])

        @pl.when(a != 0)
        def _():
            for c in range(_NCH):
                pltpu.make_async_copy(
                    const_hbm.at[midx.at[p, pl.ds(c * _CH, _CH)]],
                    mrow.at[pl.ds(c * _CH, _CH)], semm).start()
            pltpu.make_async_copy(tab_hbm.at[pl.ds(0, _GL)], mrow, semm).wait()
            for k in range(_G):
                pltpu.sync_copy(mrow.at[pl.ds(k * _E, _E)],
                                out_hbm.at[pl.ds((bb + k) * 2 * _E + _E, _E)])

        for k in range(_G):
            pltpu.sync_copy(buf.at[p, pl.ds(k * _E, _E)],
                            out_hbm.at[pl.ds((bb + k) * 2 * _E, _E)])

    prep(0, 0)

    def outer(u, c):
        g = u * 2
        prep(g + 1, 1)
        outcopy(g, 0)

        @pl.when(g + 2 < _NGRP)
        def _():
            prep(g + 2, 0)

        outcopy(g + 1, 1)
        return c

    lax.fori_loop(0, _NGRP // 2, outer, 0)


def kernel(x, entity_vectors):
    const_tab = jnp.concatenate(
        [jnp.zeros((1, _D), jnp.float32), jnp.ones((1, _D), jnp.float32)])
    mesh = plsc.VectorSubcoreMesh(core_axis_name="c", subcore_axis_name="s")
    run = pl.kernel(
        _body,
        out_type=jax.ShapeDtypeStruct((_BATCH * 2 * _E, _D), jnp.float32),
        mesh=mesh,
        compiler_params=pltpu.CompilerParams(use_tc_tiling_on_sc=False),
        scratch_types=[
            pltpu.VMEM((2, _GL), jnp.int32),
            pltpu.VMEM((2, _GL), jnp.int32),
            pltpu.VMEM((2, _GL), jnp.int32),
            pltpu.VMEM((2, _GL, _D), jnp.float32),
            pltpu.VMEM((_GL, _D), jnp.float32),
            pltpu.VMEM((_E, _D), jnp.float32),
            pltpu.SMEM((2,), jnp.int32),
            pltpu.SemaphoreType.DMA,
            pltpu.SemaphoreType.DMA,
            pltpu.SemaphoreType.DMA,
        ],
    )
    out = run(x.reshape(-1), entity_vectors, const_tab)
    return out.reshape(_BATCH, 2, _E, _D)
